# baseline (speedup 1.0000x reference)
import functools

import jax
import jax.numpy as jnp
import numpy as np
from jax import lax
from jax.experimental import pallas as pl
from jax.experimental.pallas import tpu as pltpu, tpu_sc as plsc

N = 10000
D = 128
E = 320000
EPS = 1e-5

NC = 2
NS = 16
NW = NC * NS

CH = 128
NPE = 10112
CPE = 81
ROWS_E = CPE * NW

_mesh = plsc.VectorSubcoreMesh(core_axis_name="c", subcore_axis_name="s")


@functools.partial(
    pl.kernel,
    out_type=jax.ShapeDtypeStruct((NC, NPE), jnp.float32),
    mesh=_mesh,
    scratch_types=[
        pltpu.VMEM((CPE, CH), jnp.int32),
        pltpu.VMEM((CH,), jnp.float32),
        pltpu.VMEM_SHARED((NPE,), jnp.float32),
    ],
    compiler_params=pltpu.CompilerParams(use_tc_tiling_on_sc=False),
)
def _deg_pass(dst_hbm, zeros_hbm, ones_hbm, out_hbm, dst_v, ones_v, acc_sh):
    c = lax.axis_index("c")
    s = lax.axis_index("s")
    w = c * NS + s
    sl = NPE // NS
    pltpu.sync_copy(zeros_hbm.at[pl.ds(s * sl, sl)], acc_sh.at[pl.ds(s * sl, sl)])
    pltpu.sync_copy(ones_hbm, ones_v)
    pltpu.sync_copy(dst_hbm.at[pl.ds(w * CPE, CPE)], dst_v)
    plsc.subcore_barrier()

    def body(j, carry):
        pltpu.sync_copy(ones_v, acc_sh.at[dst_v.at[j]], add=True)
        return carry

    lax.fori_loop(0, CPE, body, 0)
    plsc.subcore_barrier()
    pltpu.sync_copy(acc_sh.at[pl.ds(s * sl, sl)], out_hbm.at[c, pl.ds(s * sl, sl)])


NB = 3
NR = CPE // NB


@functools.partial(
    pl.kernel,
    out_type=jax.ShapeDtypeStruct((NC, NPE, D), jnp.float32),
    mesh=_mesh,
    scratch_types=(
        [pltpu.VMEM((CH,), jnp.int32)] * (2 * NB)
        + [pltpu.VMEM((CH, D), jnp.float32)] * NB
        + [pltpu.VMEM_SHARED((NPE, D), jnp.float32)]
        + [pltpu.SemaphoreType.DMA] * (4 * NB)
    ),
    compiler_params=pltpu.CompilerParams(use_tc_tiling_on_sc=False),
)
def _edge_pass(u_hbm, src_hbm, dst_hbm, zeros_hbm, out_hbm, *scratch):
    srcb = scratch[0:NB]
    dstb = scratch[NB:2 * NB]
    rows = scratch[2 * NB:3 * NB]
    acc_sh = scratch[3 * NB]
    lss = scratch[3 * NB + 1:3 * NB + 1 + NB]
    lds = scratch[3 * NB + 1 + NB:3 * NB + 1 + 2 * NB]
    gsem = scratch[3 * NB + 1 + 2 * NB:3 * NB + 1 + 3 * NB]
    ssem = scratch[3 * NB + 1 + 3 * NB:]
    c = lax.axis_index("c")
    s = lax.axis_index("s")
    w = c * NS + s
    sl = NPE // NS
    pltpu.sync_copy(zeros_hbm.at[pl.ds(s * sl, sl)], acc_sh.at[pl.ds(s * sl, sl)])
    plsc.subcore_barrier()

    def i_src(b, j):
        return pltpu.make_async_copy(src_hbm.at[w * CPE + j], srcb[b], lss[b])

    def i_dst(b, j):
        return pltpu.make_async_copy(dst_hbm.at[w * CPE + j], dstb[b], lds[b])

    def g_desc(b):
        return pltpu.make_async_copy(u_hbm.at[srcb[b]], rows[b], gsem[b])

    def s_desc(b):
        return pltpu.make_async_copy(rows[b], acc_sh.at[dstb[b]], ssem[b])

    for b in range(NB):
        i_src(b, b).start()
        i_dst(b, b).start()
    for b in range(NB):
        i_src(b, b).wait()
        g_desc(b).start()

    def round_(r, carry):
        j0 = r * NB
        for b in range(NB):
            g_desc(b).wait()
            i_dst(b, j0 + b).wait()
            s_desc(b).start(add=True)
            i_src(b, j0 + NB + b).start()
        for b in range(NB):
            s_desc(b).wait()
            i_dst(b, j0 + NB + b).start()
            i_src(b, j0 + NB + b).wait()
            g_desc(b).start()
        return carry

    lax.fori_loop(0, NR - 1, round_, 0)
    j0 = (NR - 1) * NB
    for b in range(NB):
        g_desc(b).wait()
        i_dst(b, j0 + b).wait()
        s_desc(b).start(add=True)
    for b in range(NB):
        s_desc(b).wait()
    plsc.subcore_barrier()
    pltpu.sync_copy(acc_sh.at[pl.ds(s * sl, sl)], out_hbm.at[c, pl.ds(s * sl, sl)])


_R = 1000


def _mm_scale_body(x_ref, w_ref, da_ref, db_ref, u_ref):
    t = jnp.dot(x_ref[...], w_ref[...], preferred_element_type=jnp.float32)
    dinv = lax.rsqrt(da_ref[...] + db_ref[...] + 1.0)
    u_ref[...] = t * dinv


def _mm_scale(x, w, da, db):
    return pl.pallas_call(
        _mm_scale_body,
        out_shape=jax.ShapeDtypeStruct((N, D), jnp.float32),
        grid=(N // _R,),
        in_specs=[
            pl.BlockSpec((_R, D), lambda i: (i, 0)),
            pl.BlockSpec((D, D), lambda i: (0, 0)),
            pl.BlockSpec((_R, 1), lambda i: (i, 0)),
            pl.BlockSpec((_R, 1), lambda i: (i, 0)),
        ],
        out_specs=pl.BlockSpec((_R, D), lambda i: (i, 0)),
    )(x, w, da, db)


def _norm_relu(sa, sb, u, da, db, b, g, be):
    dinv = lax.rsqrt(da + db + 1.0)
    t = dinv * (sa + sb + u) + b
    mean = jnp.mean(t, axis=-1, keepdims=True)
    var = jnp.mean((t - mean) ** 2, axis=-1, keepdims=True)
    t = (t - mean) * lax.rsqrt(var + EPS) * g + be
    return jnp.maximum(t, 0.0)


def _mid_body(sa_ref, sb_ref, u_ref, da_ref, db_ref, b_ref, g_ref, be_ref,
              w_ref, out_ref):
    h = _norm_relu(sa_ref[0], sb_ref[0], u_ref[...], da_ref[...],
                   db_ref[...], b_ref[...], g_ref[...], be_ref[...])
    dinv = lax.rsqrt(da_ref[...] + db_ref[...] + 1.0)
    out_ref[...] = jnp.dot(h, w_ref[...], preferred_element_type=jnp.float32) * dinv


_SA = pl.BlockSpec((1, _R, D), lambda i: (0, i, 0))
_SB = pl.BlockSpec((1, _R, D), lambda i: (1, i, 0))


def _mid_stage(s, u, da, db, b, g, be, w):
    vec = pl.BlockSpec((1, D), lambda i: (0, 0))
    row = pl.BlockSpec((_R, D), lambda i: (i, 0))
    col = pl.BlockSpec((_R, 1), lambda i: (i, 0))
    return pl.pallas_call(
        _mid_body,
        out_shape=jax.ShapeDtypeStruct((N, D), jnp.float32),
        grid=(N // _R,),
        in_specs=[_SA, _SB, row, col, col, vec, vec, vec,
                  pl.BlockSpec((D, D), lambda i: (0, 0))],
        out_specs=row,
    )(s, s, u, da, db, b, g, be, w)


def _final_body(sa_ref, sb_ref, u_ref, da_ref, db_ref, b_ref, g_ref, be_ref,
                out_ref):
    out_ref[...] = _norm_relu(sa_ref[0], sb_ref[0], u_ref[...], da_ref[...],
                              db_ref[...], b_ref[...], g_ref[...], be_ref[...])


def _final_stage(s, u, da, db, b, g, be):
    vec = pl.BlockSpec((1, D), lambda i: (0, 0))
    row = pl.BlockSpec((_R, D), lambda i: (i, 0))
    col = pl.BlockSpec((_R, 1), lambda i: (i, 0))
    return pl.pallas_call(
        _final_body,
        out_shape=jax.ShapeDtypeStruct((N, D), jnp.float32),
        grid=(N // _R,),
        in_specs=[_SA, _SB, row, col, col, vec, vec, vec],
        out_specs=row,
    )(s, s, u, da, db, b, g, be)


def kernel(x, edge_index, W0, b0, g0, be0, W1, b1, g1, be1):
    src = edge_index[0].astype(jnp.int32)
    dst = edge_index[1].astype(jnp.int32)
    pad_e = ROWS_E * CH - E
    pad_src_e = (jnp.arange(pad_e, dtype=jnp.int32) * 37) % N
    pad_dst_e = N + jnp.arange(pad_e, dtype=jnp.int32) % (NPE - N)
    src3d = jnp.concatenate([src, pad_src_e]).reshape(ROWS_E, CH)
    dst3d = jnp.concatenate([dst, pad_dst_e]).reshape(ROWS_E, CH)
    zeros2d = jnp.zeros((NPE, D), jnp.float32)
    zeros1d = jnp.zeros((NPE,), jnp.float32)
    ones_ch = jnp.ones((CH,), jnp.float32)

    degs = _deg_pass(dst3d, zeros1d, ones_ch)
    da = degs[0, :N][:, None]
    db = degs[1, :N][:, None]

    b0r, g0r, be0r = b0.reshape(1, D), g0.reshape(1, D), be0.reshape(1, D)
    b1r, g1r, be1r = b1.reshape(1, D), g1.reshape(1, D), be1.reshape(1, D)

    u0 = _mm_scale(x, W0, da, db)
    s0 = _edge_pass(u0, src3d, dst3d, zeros2d)
    u1 = _mid_stage(s0, u0, da, db, b0r, g0r, be0r, W1)
    s1 = _edge_pass(u1, src3d, dst3d, zeros2d)
    return _final_stage(s1, u1, da, db, b1r, g1r, be1r)


# speedup vs baseline: 1.0435x; 1.0435x over previous
import functools

import jax
import jax.numpy as jnp
import numpy as np
from jax import lax
from jax.experimental import pallas as pl
from jax.experimental.pallas import tpu as pltpu, tpu_sc as plsc

N = 10000
D = 128
E = 320000
EPS = 1e-5

NC = 2
NS = 16
NW = NC * NS

CH = 96
NPE = 10112
CPE = 108
ROWS_E = CPE * NW

_mesh = plsc.VectorSubcoreMesh(core_axis_name="c", subcore_axis_name="s")


@functools.partial(
    pl.kernel,
    out_type=jax.ShapeDtypeStruct((NC, NPE), jnp.float32),
    mesh=_mesh,
    scratch_types=[
        pltpu.VMEM((CPE, CH), jnp.int32),
        pltpu.VMEM((CH,), jnp.float32),
        pltpu.VMEM_SHARED((NPE,), jnp.float32),
    ],
    compiler_params=pltpu.CompilerParams(use_tc_tiling_on_sc=False),
)
def _deg_pass(dst_hbm, zeros_hbm, ones_hbm, out_hbm, dst_v, ones_v, acc_sh):
    c = lax.axis_index("c")
    s = lax.axis_index("s")
    w = c * NS + s
    sl = NPE // NS
    pltpu.sync_copy(zeros_hbm.at[pl.ds(s * sl, sl)], acc_sh.at[pl.ds(s * sl, sl)])
    pltpu.sync_copy(ones_hbm, ones_v)
    pltpu.sync_copy(dst_hbm.at[pl.ds(w * CPE, CPE)], dst_v)
    plsc.subcore_barrier()

    def body(j, carry):
        pltpu.sync_copy(ones_v, acc_sh.at[dst_v.at[j]], add=True)
        return carry

    lax.fori_loop(0, CPE, body, 0)
    plsc.subcore_barrier()
    pltpu.sync_copy(acc_sh.at[pl.ds(s * sl, sl)], out_hbm.at[c, pl.ds(s * sl, sl)])


NB = 4
NR = CPE // NB


@functools.partial(
    pl.kernel,
    out_type=jax.ShapeDtypeStruct((NC, NPE, D), jnp.float32),
    mesh=_mesh,
    scratch_types=(
        [pltpu.VMEM((CH,), jnp.int32)] * (2 * NB)
        + [pltpu.VMEM((CH, D), jnp.float32)] * NB
        + [pltpu.VMEM_SHARED((NPE, D), jnp.float32)]
        + [pltpu.SemaphoreType.DMA] * (4 * NB)
    ),
    compiler_params=pltpu.CompilerParams(use_tc_tiling_on_sc=False),
)
def _edge_pass(u_hbm, src_hbm, dst_hbm, zeros_hbm, out_hbm, *scratch):
    srcb = scratch[0:NB]
    dstb = scratch[NB:2 * NB]
    rows = scratch[2 * NB:3 * NB]
    acc_sh = scratch[3 * NB]
    lss = scratch[3 * NB + 1:3 * NB + 1 + NB]
    lds = scratch[3 * NB + 1 + NB:3 * NB + 1 + 2 * NB]
    gsem = scratch[3 * NB + 1 + 2 * NB:3 * NB + 1 + 3 * NB]
    ssem = scratch[3 * NB + 1 + 3 * NB:]
    c = lax.axis_index("c")
    s = lax.axis_index("s")
    w = c * NS + s
    sl = NPE // NS
    pltpu.sync_copy(zeros_hbm.at[pl.ds(s * sl, sl)], acc_sh.at[pl.ds(s * sl, sl)])
    plsc.subcore_barrier()

    def i_src(b, j):
        return pltpu.make_async_copy(src_hbm.at[w * CPE + j], srcb[b], lss[b])

    def i_dst(b, j):
        return pltpu.make_async_copy(dst_hbm.at[w * CPE + j], dstb[b], lds[b])

    def g_desc(b):
        return pltpu.make_async_copy(u_hbm.at[srcb[b]], rows[b], gsem[b])

    def s_desc(b):
        return pltpu.make_async_copy(rows[b], acc_sh.at[dstb[b]], ssem[b])

    for b in range(NB):
        i_src(b, b).start()
        i_dst(b, b).start()
    for b in range(NB):
        i_src(b, b).wait()
        g_desc(b).start()

    def round_(r, carry):
        j0 = r * NB
        for b in range(NB):
            g_desc(b).wait()
            i_dst(b, j0 + b).wait()
            s_desc(b).start(add=True)
            i_src(b, j0 + NB + b).start()
        for b in range(NB):
            s_desc(b).wait()
            i_dst(b, j0 + NB + b).start()
            i_src(b, j0 + NB + b).wait()
            g_desc(b).start()
        return carry

    lax.fori_loop(0, NR - 1, round_, 0)
    j0 = (NR - 1) * NB
    for b in range(NB):
        g_desc(b).wait()
        i_dst(b, j0 + b).wait()
        s_desc(b).start(add=True)
    for b in range(NB):
        s_desc(b).wait()
    plsc.subcore_barrier()
    pltpu.sync_copy(acc_sh.at[pl.ds(s * sl, sl)], out_hbm.at[c, pl.ds(s * sl, sl)])


_R = 1000


def _mm_scale_body(x_ref, w_ref, da_ref, db_ref, u_ref):
    t = jnp.dot(x_ref[...], w_ref[...], preferred_element_type=jnp.float32)
    dinv = lax.rsqrt(da_ref[...] + db_ref[...] + 1.0)
    u_ref[...] = t * dinv


def _mm_scale(x, w, da, db):
    return pl.pallas_call(
        _mm_scale_body,
        out_shape=jax.ShapeDtypeStruct((N, D), jnp.float32),
        grid=(N // _R,),
        in_specs=[
            pl.BlockSpec((_R, D), lambda i: (i, 0)),
            pl.BlockSpec((D, D), lambda i: (0, 0)),
            pl.BlockSpec((_R, 1), lambda i: (i, 0)),
            pl.BlockSpec((_R, 1), lambda i: (i, 0)),
        ],
        out_specs=pl.BlockSpec((_R, D), lambda i: (i, 0)),
    )(x, w, da, db)


def _norm_relu(sa, sb, u, da, db, b, g, be):
    dinv = lax.rsqrt(da + db + 1.0)
    t = dinv * (sa + sb + u) + b
    mean = jnp.mean(t, axis=-1, keepdims=True)
    var = jnp.mean((t - mean) ** 2, axis=-1, keepdims=True)
    t = (t - mean) * lax.rsqrt(var + EPS) * g + be
    return jnp.maximum(t, 0.0)


def _mid_body(sa_ref, sb_ref, u_ref, da_ref, db_ref, b_ref, g_ref, be_ref,
              w_ref, out_ref):
    h = _norm_relu(sa_ref[0], sb_ref[0], u_ref[...], da_ref[...],
                   db_ref[...], b_ref[...], g_ref[...], be_ref[...])
    dinv = lax.rsqrt(da_ref[...] + db_ref[...] + 1.0)
    out_ref[...] = jnp.dot(h, w_ref[...], preferred_element_type=jnp.float32) * dinv


_SA = pl.BlockSpec((1, _R, D), lambda i: (0, i, 0))
_SB = pl.BlockSpec((1, _R, D), lambda i: (1, i, 0))


def _mid_stage(s, u, da, db, b, g, be, w):
    vec = pl.BlockSpec((1, D), lambda i: (0, 0))
    row = pl.BlockSpec((_R, D), lambda i: (i, 0))
    col = pl.BlockSpec((_R, 1), lambda i: (i, 0))
    return pl.pallas_call(
        _mid_body,
        out_shape=jax.ShapeDtypeStruct((N, D), jnp.float32),
        grid=(N // _R,),
        in_specs=[_SA, _SB, row, col, col, vec, vec, vec,
                  pl.BlockSpec((D, D), lambda i: (0, 0))],
        out_specs=row,
    )(s, s, u, da, db, b, g, be, w)


def _final_body(sa_ref, sb_ref, u_ref, da_ref, db_ref, b_ref, g_ref, be_ref,
                out_ref):
    out_ref[...] = _norm_relu(sa_ref[0], sb_ref[0], u_ref[...], da_ref[...],
                              db_ref[...], b_ref[...], g_ref[...], be_ref[...])


def _final_stage(s, u, da, db, b, g, be):
    vec = pl.BlockSpec((1, D), lambda i: (0, 0))
    row = pl.BlockSpec((_R, D), lambda i: (i, 0))
    col = pl.BlockSpec((_R, 1), lambda i: (i, 0))
    return pl.pallas_call(
        _final_body,
        out_shape=jax.ShapeDtypeStruct((N, D), jnp.float32),
        grid=(N // _R,),
        in_specs=[_SA, _SB, row, col, col, vec, vec, vec],
        out_specs=row,
    )(s, s, u, da, db, b, g, be)


def kernel(x, edge_index, W0, b0, g0, be0, W1, b1, g1, be1):
    src = edge_index[0].astype(jnp.int32)
    dst = edge_index[1].astype(jnp.int32)
    pad_e = ROWS_E * CH - E
    pad_src_e = (jnp.arange(pad_e, dtype=jnp.int32) * 37) % N
    pad_dst_e = N + jnp.arange(pad_e, dtype=jnp.int32) % (NPE - N)
    src3d = jnp.concatenate([src, pad_src_e]).reshape(ROWS_E, CH)
    dst3d = jnp.concatenate([dst, pad_dst_e]).reshape(ROWS_E, CH)
    zeros2d = jnp.zeros((NPE, D), jnp.float32)
    zeros1d = jnp.zeros((NPE,), jnp.float32)
    ones_ch = jnp.ones((CH,), jnp.float32)

    degs = _deg_pass(dst3d, zeros1d, ones_ch)
    da = degs[0, :N][:, None]
    db = degs[1, :N][:, None]

    b0r, g0r, be0r = b0.reshape(1, D), g0.reshape(1, D), be0.reshape(1, D)
    b1r, g1r, be1r = b1.reshape(1, D), g1.reshape(1, D), be1.reshape(1, D)

    u0 = _mm_scale(x, W0, da, db)
    s0 = _edge_pass(u0, src3d, dst3d, zeros2d)
    u1 = _mid_stage(s0, u0, da, db, b0r, g0r, be0r, W1)
    s1 = _edge_pass(u1, src3d, dst3d, zeros2d)
    return _final_stage(s1, u1, da, db, b1r, g1r, be1r)


# speedup vs baseline: 1.0690x; 1.0245x over previous
import functools

import jax
import jax.numpy as jnp
import numpy as np
from jax import lax
from jax.experimental import pallas as pl
from jax.experimental.pallas import tpu as pltpu, tpu_sc as plsc

N = 10000
D = 128
E = 320000
EPS = 1e-5

NC = 2
NS = 16
NW = NC * NS

CH = 64
NPE = 10112
CPE = 162
ROWS_E = CPE * NW

_mesh = plsc.VectorSubcoreMesh(core_axis_name="c", subcore_axis_name="s")


@functools.partial(
    pl.kernel,
    out_type=jax.ShapeDtypeStruct((NC, NPE), jnp.float32),
    mesh=_mesh,
    scratch_types=[
        pltpu.VMEM((CPE, CH), jnp.int32),
        pltpu.VMEM((CH,), jnp.float32),
        pltpu.VMEM_SHARED((NPE,), jnp.float32),
    ],
    compiler_params=pltpu.CompilerParams(use_tc_tiling_on_sc=False),
)
def _deg_pass(dst_hbm, zeros_hbm, ones_hbm, out_hbm, dst_v, ones_v, acc_sh):
    c = lax.axis_index("c")
    s = lax.axis_index("s")
    w = c * NS + s
    sl = NPE // NS
    pltpu.sync_copy(zeros_hbm.at[pl.ds(s * sl, sl)], acc_sh.at[pl.ds(s * sl, sl)])
    pltpu.sync_copy(ones_hbm, ones_v)
    pltpu.sync_copy(dst_hbm.at[pl.ds(w * CPE, CPE)], dst_v)
    plsc.subcore_barrier()

    def body(j, carry):
        pltpu.sync_copy(ones_v, acc_sh.at[dst_v.at[j]], add=True)
        return carry

    lax.fori_loop(0, CPE, body, 0)
    plsc.subcore_barrier()
    pltpu.sync_copy(acc_sh.at[pl.ds(s * sl, sl)], out_hbm.at[c, pl.ds(s * sl, sl)])


NB = 6
NR = CPE // NB


@functools.partial(
    pl.kernel,
    out_type=jax.ShapeDtypeStruct((NC, NPE, D), jnp.float32),
    mesh=_mesh,
    scratch_types=(
        [pltpu.VMEM((CH,), jnp.int32)] * (2 * NB)
        + [pltpu.VMEM((CH, D), jnp.float32)] * NB
        + [pltpu.VMEM_SHARED((NPE, D), jnp.float32)]
        + [pltpu.SemaphoreType.DMA] * (4 * NB)
    ),
    compiler_params=pltpu.CompilerParams(use_tc_tiling_on_sc=False),
)
def _edge_pass(u_hbm, src_hbm, dst_hbm, zeros_hbm, out_hbm, *scratch):
    srcb = scratch[0:NB]
    dstb = scratch[NB:2 * NB]
    rows = scratch[2 * NB:3 * NB]
    acc_sh = scratch[3 * NB]
    lss = scratch[3 * NB + 1:3 * NB + 1 + NB]
    lds = scratch[3 * NB + 1 + NB:3 * NB + 1 + 2 * NB]
    gsem = scratch[3 * NB + 1 + 2 * NB:3 * NB + 1 + 3 * NB]
    ssem = scratch[3 * NB + 1 + 3 * NB:]
    c = lax.axis_index("c")
    s = lax.axis_index("s")
    w = c * NS + s
    sl = NPE // NS
    pltpu.sync_copy(zeros_hbm.at[pl.ds(s * sl, sl)], acc_sh.at[pl.ds(s * sl, sl)])
    plsc.subcore_barrier()

    def i_src(b, j):
        return pltpu.make_async_copy(src_hbm.at[w * CPE + j], srcb[b], lss[b])

    def i_dst(b, j):
        return pltpu.make_async_copy(dst_hbm.at[w * CPE + j], dstb[b], lds[b])

    def g_desc(b):
        return pltpu.make_async_copy(u_hbm.at[srcb[b]], rows[b], gsem[b])

    def s_desc(b):
        return pltpu.make_async_copy(rows[b], acc_sh.at[dstb[b]], ssem[b])

    for b in range(NB):
        i_src(b, b).start()
        i_dst(b, b).start()
    for b in range(NB):
        i_src(b, b).wait()
        g_desc(b).start()

    def round_(r, carry):
        j0 = r * NB
        for b in range(NB):
            g_desc(b).wait()
            i_dst(b, j0 + b).wait()
            s_desc(b).start(add=True)
            i_src(b, j0 + NB + b).start()
        for b in range(NB):
            s_desc(b).wait()
            i_dst(b, j0 + NB + b).start()
            i_src(b, j0 + NB + b).wait()
            g_desc(b).start()
        return carry

    lax.fori_loop(0, NR - 1, round_, 0)
    j0 = (NR - 1) * NB
    for b in range(NB):
        g_desc(b).wait()
        i_dst(b, j0 + b).wait()
        s_desc(b).start(add=True)
    for b in range(NB):
        s_desc(b).wait()
    plsc.subcore_barrier()
    pltpu.sync_copy(acc_sh.at[pl.ds(s * sl, sl)], out_hbm.at[c, pl.ds(s * sl, sl)])


_R = 1000


def _mm_scale_body(x_ref, w_ref, da_ref, db_ref, u_ref):
    t = jnp.dot(x_ref[...], w_ref[...], preferred_element_type=jnp.float32)
    dinv = lax.rsqrt(da_ref[...] + db_ref[...] + 1.0)
    u_ref[...] = t * dinv


def _mm_scale(x, w, da, db):
    return pl.pallas_call(
        _mm_scale_body,
        out_shape=jax.ShapeDtypeStruct((N, D), jnp.float32),
        grid=(N // _R,),
        in_specs=[
            pl.BlockSpec((_R, D), lambda i: (i, 0)),
            pl.BlockSpec((D, D), lambda i: (0, 0)),
            pl.BlockSpec((_R, 1), lambda i: (i, 0)),
            pl.BlockSpec((_R, 1), lambda i: (i, 0)),
        ],
        out_specs=pl.BlockSpec((_R, D), lambda i: (i, 0)),
    )(x, w, da, db)


def _norm_relu(sa, sb, u, da, db, b, g, be):
    dinv = lax.rsqrt(da + db + 1.0)
    t = dinv * (sa + sb + u) + b
    mean = jnp.mean(t, axis=-1, keepdims=True)
    var = jnp.mean((t - mean) ** 2, axis=-1, keepdims=True)
    t = (t - mean) * lax.rsqrt(var + EPS) * g + be
    return jnp.maximum(t, 0.0)


def _mid_body(sa_ref, sb_ref, u_ref, da_ref, db_ref, b_ref, g_ref, be_ref,
              w_ref, out_ref):
    h = _norm_relu(sa_ref[0], sb_ref[0], u_ref[...], da_ref[...],
                   db_ref[...], b_ref[...], g_ref[...], be_ref[...])
    dinv = lax.rsqrt(da_ref[...] + db_ref[...] + 1.0)
    out_ref[...] = jnp.dot(h, w_ref[...], preferred_element_type=jnp.float32) * dinv


_SA = pl.BlockSpec((1, _R, D), lambda i: (0, i, 0))
_SB = pl.BlockSpec((1, _R, D), lambda i: (1, i, 0))


def _mid_stage(s, u, da, db, b, g, be, w):
    vec = pl.BlockSpec((1, D), lambda i: (0, 0))
    row = pl.BlockSpec((_R, D), lambda i: (i, 0))
    col = pl.BlockSpec((_R, 1), lambda i: (i, 0))
    return pl.pallas_call(
        _mid_body,
        out_shape=jax.ShapeDtypeStruct((N, D), jnp.float32),
        grid=(N // _R,),
        in_specs=[_SA, _SB, row, col, col, vec, vec, vec,
                  pl.BlockSpec((D, D), lambda i: (0, 0))],
        out_specs=row,
    )(s, s, u, da, db, b, g, be, w)


def _final_body(sa_ref, sb_ref, u_ref, da_ref, db_ref, b_ref, g_ref, be_ref,
                out_ref):
    out_ref[...] = _norm_relu(sa_ref[0], sb_ref[0], u_ref[...], da_ref[...],
                              db_ref[...], b_ref[...], g_ref[...], be_ref[...])


def _final_stage(s, u, da, db, b, g, be):
    vec = pl.BlockSpec((1, D), lambda i: (0, 0))
    row = pl.BlockSpec((_R, D), lambda i: (i, 0))
    col = pl.BlockSpec((_R, 1), lambda i: (i, 0))
    return pl.pallas_call(
        _final_body,
        out_shape=jax.ShapeDtypeStruct((N, D), jnp.float32),
        grid=(N // _R,),
        in_specs=[_SA, _SB, row, col, col, vec, vec, vec],
        out_specs=row,
    )(s, s, u, da, db, b, g, be)


def kernel(x, edge_index, W0, b0, g0, be0, W1, b1, g1, be1):
    src = edge_index[0].astype(jnp.int32)
    dst = edge_index[1].astype(jnp.int32)
    pad_e = ROWS_E * CH - E
    pad_src_e = (jnp.arange(pad_e, dtype=jnp.int32) * 37) % N
    pad_dst_e = N + jnp.arange(pad_e, dtype=jnp.int32) % (NPE - N)
    src3d = jnp.concatenate([src, pad_src_e]).reshape(ROWS_E, CH)
    dst3d = jnp.concatenate([dst, pad_dst_e]).reshape(ROWS_E, CH)
    zeros2d = jnp.zeros((NPE, D), jnp.float32)
    zeros1d = jnp.zeros((NPE,), jnp.float32)
    ones_ch = jnp.ones((CH,), jnp.float32)

    degs = _deg_pass(dst3d, zeros1d, ones_ch)
    da = degs[0, :N][:, None]
    db = degs[1, :N][:, None]

    b0r, g0r, be0r = b0.reshape(1, D), g0.reshape(1, D), be0.reshape(1, D)
    b1r, g1r, be1r = b1.reshape(1, D), g1.reshape(1, D), be1.reshape(1, D)

    u0 = _mm_scale(x, W0, da, db)
    s0 = _edge_pass(u0, src3d, dst3d, zeros2d)
    u1 = _mid_stage(s0, u0, da, db, b0r, g0r, be0r, W1)
    s1 = _edge_pass(u1, src3d, dst3d, zeros2d)
    return _final_stage(s1, u1, da, db, b1r, g1r, be1r)
